# Initial kernel scaffold; baseline (speedup 1.0000x reference)
import jax
import jax.numpy as jnp
from jax.experimental import pallas as pl


def kernel(x, edge_index, Wl1, bl1, Wr1, Wl2, bl2, Wr2, g1, b1, g2, b2, Wo, bo):
    raise NotImplementedError("write your pallas kernel here")


# speedup vs baseline: 3.9313x; 3.9313x over previous
import functools

import jax
import jax.numpy as jnp
from jax import lax
from jax.experimental import pallas as pl
from jax.experimental.pallas import tpu as pltpu
from jax.experimental.pallas import tpu_sc as plsc

N = 10000
E = 320000
D = 128

NC = 2
NS = 16
NW = NC * NS

NP = 10240
EP = 327680
EPW = EP // NW
CH = 128
NCHUNK = EPW // CH
RPT = NP // NS
NSTRIPE = RPT // CH


def _zero_stripe(sh, stage_v, r0):
    for k in range(NSTRIPE):
        pltpu.sync_copy(stage_v, sh.at[pl.ds(r0 + k * CH, CH)])


def _writeback_stripe(sh, stage_v, out, r0, c):
    for k in range(NSTRIPE):
        pltpu.sync_copy(sh.at[pl.ds(r0 + k * CH, CH)], stage_v)
        pltpu.sync_copy(stage_v, out.at[pl.ds(c * NP + r0 + k * CH, CH)])


def _sc_agg_body(table, srcm, dstm, zrows, agg_out,
                 src_v, dst_v, rows_v, agg_sh, sem):
    c = lax.axis_index("c")
    s = lax.axis_index("s")
    wid = s * NC + c
    r0 = s * RPT
    pltpu.sync_copy(zrows, rows_v)
    _zero_stripe(agg_sh, rows_v, r0)
    plsc.subcore_barrier()

    base = wid * EPW

    def chunk(j, carry):
        pltpu.sync_copy(srcm.at[pl.ds(base + j * CH, CH)], src_v)
        pltpu.sync_copy(dstm.at[pl.ds(base + j * CH, CH)], dst_v)
        pltpu.async_copy(table.at[src_v], rows_v, sem).wait()
        pltpu.sync_copy(rows_v, agg_sh.at[dst_v], add=True)
        return carry

    lax.fori_loop(0, NCHUNK, chunk, 0)
    plsc.subcore_barrier()
    _writeback_stripe(agg_sh, rows_v, agg_out, r0, c)


def _sc_cnt_body(dstm, zrows, ones, cnt_out, dst_v, rows_v, cnt_sh):
    c = lax.axis_index("c")
    s = lax.axis_index("s")
    wid = s * NC + c
    r0 = s * RPT
    pltpu.sync_copy(zrows, rows_v)
    _zero_stripe(cnt_sh, rows_v, r0)
    pltpu.sync_copy(ones, rows_v)
    plsc.subcore_barrier()

    base = wid * EPW

    def chunk(j, carry):
        pltpu.sync_copy(dstm.at[pl.ds(base + j * CH, CH)], dst_v)
        pltpu.sync_copy(rows_v, cnt_sh.at[dst_v], add=True)
        return carry

    lax.fori_loop(0, NCHUNK, chunk, 0)
    plsc.subcore_barrier()
    _writeback_stripe(cnt_sh, rows_v, cnt_out, r0, c)


def _sc_mesh():
    return plsc.VectorSubcoreMesh(core_axis_name="c", subcore_axis_name="s")


@functools.lru_cache(maxsize=None)
def _sc_agg():
    return pl.kernel(
        _sc_agg_body,
        out_type=jax.ShapeDtypeStruct((NC * NP, D), jnp.float32),
        mesh=_sc_mesh(),
        scratch_types=[
            pltpu.VMEM((CH,), jnp.int32),
            pltpu.VMEM((CH,), jnp.int32),
            pltpu.VMEM((CH, D), jnp.float32),
            pltpu.VMEM_SHARED((NP, D), jnp.float32),
            pltpu.SemaphoreType.DMA,
        ],
    )


@functools.lru_cache(maxsize=None)
def _sc_cnt():
    return pl.kernel(
        _sc_cnt_body,
        out_type=jax.ShapeDtypeStruct((NC * NP, D), jnp.float32),
        mesh=_sc_mesh(),
        scratch_types=[
            pltpu.VMEM((CH,), jnp.int32),
            pltpu.VMEM((CH, D), jnp.float32),
            pltpu.VMEM_SHARED((NP, D), jnp.float32),
        ],
    )


_BLK = 640
_GRID = NP // _BLK

_DN = (((1,), (1,)), ((), ()))


def _tc_layer1_body(x_r, aggA_r, aggB_r, cntA_r, cntB_r, wl_r, wr_r, bl_r,
                    g_r, b_r, o_r):
    x = x_r[...]
    cnt = cntA_r[:, :1] + cntB_r[:, :1] + 1.0
    mean = (aggA_r[...] + aggB_r[...] + x) / cnt
    h = lax.dot_general(mean, wl_r[...], _DN, preferred_element_type=jnp.float32)
    h = h + lax.dot_general(x, wr_r[...], _DN, preferred_element_type=jnp.float32)
    h = jnp.maximum(h + bl_r[...], 0.0)
    mu = jnp.mean(h, axis=1, keepdims=True)
    var = jnp.mean((h - mu) * (h - mu), axis=1, keepdims=True)
    o_r[...] = (h - mu) * lax.rsqrt(var + 1e-5) * g_r[...] + b_r[...]


def _tc_layer2_body(h1_r, aggA_r, aggB_r, cntA_r, cntB_r, wl_r, wr_r, bl_r,
                    g_r, b_r, wo_r, bo_r, o_r):
    h1 = h1_r[...]
    cnt = cntA_r[:, :1] + cntB_r[:, :1] + 1.0
    mean = (aggA_r[...] + aggB_r[...] + h1) / cnt
    h = lax.dot_general(mean, wl_r[...], _DN, preferred_element_type=jnp.float32)
    h = h + lax.dot_general(h1, wr_r[...], _DN, preferred_element_type=jnp.float32)
    h = jnp.maximum(h + bl_r[...], 0.0)
    mu = jnp.mean(h, axis=1, keepdims=True)
    var = jnp.mean((h - mu) * (h - mu), axis=1, keepdims=True)
    h = (h - mu) * lax.rsqrt(var + 1e-5) * g_r[...] + b_r[...]
    o = lax.dot_general(h, wo_r[...], _DN, preferred_element_type=jnp.float32)
    o = o + bo_r[...]
    m = jnp.max(o, axis=1, keepdims=True)
    lse = m + jnp.log(jnp.sum(jnp.exp(o - m), axis=1, keepdims=True))
    o_r[...] = o - lse


def _row_spec(i_off=0, w=D):
    return pl.BlockSpec((_BLK, w), lambda i, o=i_off: (i + o, 0))


def _full_spec(r, c):
    return pl.BlockSpec((r, c), lambda i: (0, 0))


_tc_layer1 = pl.pallas_call(
    _tc_layer1_body,
    grid=(_GRID,),
    in_specs=[
        _row_spec(), _row_spec(), _row_spec(_GRID),
        _row_spec(), _row_spec(_GRID),
        _full_spec(D, D), _full_spec(D, D),
        _full_spec(1, D), _full_spec(1, D), _full_spec(1, D),
    ],
    out_specs=_row_spec(),
    out_shape=jax.ShapeDtypeStruct((NP, D), jnp.float32),
)

_tc_layer2 = pl.pallas_call(
    _tc_layer2_body,
    grid=(_GRID,),
    in_specs=[
        _row_spec(), _row_spec(), _row_spec(_GRID),
        _row_spec(), _row_spec(_GRID),
        _full_spec(D, D), _full_spec(D, D),
        _full_spec(1, D), _full_spec(1, D), _full_spec(1, D),
        _full_spec(D, D), _full_spec(1, D),
    ],
    out_specs=_row_spec(),
    out_shape=jax.ShapeDtypeStruct((NP, D), jnp.float32),
)


def kernel(x, edge_index, Wl1, bl1, Wr1, Wl2, bl2, Wr2, g1, b1, g2, b2, Wo, bo):
    f32 = jnp.float32
    x_pad = jnp.pad(x, ((0, NP - N), (0, 0)))
    pad = EP - E
    srcm = jnp.concatenate([edge_index[0], jnp.zeros((pad,), jnp.int32)])
    dstm = jnp.concatenate([edge_index[1], jnp.full((pad,), N, jnp.int32)])
    zrows = jnp.zeros((CH, D), f32)
    ones = jnp.ones((CH, D), f32)

    cnt = _sc_cnt()(dstm, zrows, ones)
    agg1 = _sc_agg()(x_pad, srcm, dstm, zrows)
    bl1r, g1r, b1r = bl1.reshape(1, D), g1.reshape(1, D), b1.reshape(1, D)
    h1 = _tc_layer1(x_pad, agg1, agg1, cnt, cnt, Wl1, Wr1, bl1r, g1r, b1r)

    agg2 = _sc_agg()(h1, srcm, dstm, zrows)
    bl2r, g2r, b2r = bl2.reshape(1, D), g2.reshape(1, D), b2.reshape(1, D)
    bor = bo.reshape(1, D)
    out = _tc_layer2(h1, agg2, agg2, cnt, cnt, Wl2, Wr2, bl2r, g2r, b2r,
                     Wo, bor)
    return out[:N]
